# Initial kernel scaffold; baseline (speedup 1.0000x reference)
import functools

import jax
import jax.numpy as jnp
from jax import lax
from jax.experimental import pallas as pl
from jax.experimental.pallas import tpu as pltpu
from jax.experimental.pallas import tpu_sc as plsc

N = 10000
E = 320000
D = 128
NORM_FACTOR = 100.0
COORDS_RANGE = 15.0
NORM_CONSTANT = 1.0

NC = 2
NS = 16
NW = NC * NS
EPW = E // NW
C = 80
NCH = EPW // C
RPT = N // NS

EB = 2000
NB = 1000


def _silu(v):
    return v * jax.nn.sigmoid(v)


def _make_gather(Dt):
    mesh = plsc.VectorSubcoreMesh(core_axis_name="c", subcore_axis_name="s")

    @functools.partial(
        pl.kernel,
        out_type=[jax.ShapeDtypeStruct((E, Dt), jnp.float32),
                  jax.ShapeDtypeStruct((E, Dt), jnp.float32)],
        mesh=mesh,
        scratch_types=[pltpu.VMEM((C,), jnp.int32),
                       pltpu.VMEM((C,), jnp.int32),
                       pltpu.VMEM((C, Dt), jnp.float32),
                       pltpu.VMEM((C, Dt), jnp.float32),
                       pltpu.SemaphoreType.DMA,
                       pltpu.SemaphoreType.DMA],
    )
    def gk(tab, rowi, coli, out_r, out_c, ir, ic, br, bc, s1, s2):
        wid = lax.axis_index("s") * NC + lax.axis_index("c")
        base0 = wid * EPW

        def body(i, carry):
            base = base0 + i * C
            pltpu.sync_copy(rowi.at[pl.ds(base, C)], ir)
            pltpu.sync_copy(coli.at[pl.ds(base, C)], ic)
            cp1 = pltpu.async_copy(tab.at[ir], br, s1)
            cp2 = pltpu.async_copy(tab.at[ic], bc, s2)
            cp1.wait()
            cp2.wait()
            pltpu.sync_copy(br, out_r.at[pl.ds(base, C)])
            pltpu.sync_copy(bc, out_c.at[pl.ds(base, C)])
            return carry

        lax.fori_loop(0, NCH, body, 0)

    return gk


def _make_scatter(Dv):
    mesh = plsc.VectorSubcoreMesh(core_axis_name="c", subcore_axis_name="s")

    @functools.partial(
        pl.kernel,
        out_type=jax.ShapeDtypeStruct((NC * N, Dv), jnp.float32),
        mesh=mesh,
        scratch_types=[pltpu.VMEM((C,), jnp.int32),
                       pltpu.VMEM((C, Dv), jnp.float32),
                       pltpu.VMEM((RPT, Dv), jnp.float32),
                       pltpu.VMEM_SHARED((N, Dv), jnp.float32)],
    )
    def sk(vals, rowi, zer, out, ib, vb, zb, acc):
        cid = lax.axis_index("c")
        sid = lax.axis_index("s")
        base0 = (cid * NS + sid) * EPW
        pltpu.sync_copy(zer, zb)
        pltpu.sync_copy(zb, acc.at[pl.ds(sid * RPT, RPT)])
        plsc.subcore_barrier()

        def body(i, carry):
            base = base0 + i * C
            pltpu.sync_copy(rowi.at[pl.ds(base, C)], ib)
            pltpu.sync_copy(vals.at[pl.ds(base, C)], vb)
            pltpu.sync_copy(vb, acc.at[ib], add=True)
            return carry

        lax.fori_loop(0, NCH, body, 0)
        plsc.subcore_barrier()
        pltpu.sync_copy(acc.at[pl.ds(sid * RPT, RPT)], zb)
        pltpu.sync_copy(zb, out.at[pl.ds(cid * N + sid * RPT, RPT)])

    return sk


def _edge1_body(gr, gc, ea, w1r, w1c, w1e, b1, w2, b2, watt, batt, m_out, cdr_out):
    hr = gr[:, :D]
    hc = gc[:, :D]
    xr = gr[:, D:D + 3]
    xc = gc[:, D:D + 3]
    cd = xr - xc
    radial = jnp.sum(cd * cd, axis=1, keepdims=True)
    cdn = cd / (jnp.sqrt(radial + 1e-8) + NORM_CONSTANT)
    lane0 = (lax.broadcasted_iota(jnp.int32, (1, 8), 1) == 0).astype(jnp.float32)
    eap = ea[...] + radial * lane0
    pre = (jnp.dot(hr, w1r[...], preferred_element_type=jnp.float32)
           + jnp.dot(hc, w1c[...], preferred_element_type=jnp.float32)
           + jnp.dot(eap, w1e[...], preferred_element_type=jnp.float32)
           + b1[...])
    m = _silu(pre)
    m = _silu(jnp.dot(m, w2[...], preferred_element_type=jnp.float32) + b2[...])
    att = jax.nn.sigmoid(jnp.sum(m * watt[...], axis=1, keepdims=True) + batt[...])
    m_out[...] = m * att
    cdr_out[...] = jnp.concatenate([cdn, radial], axis=1)


def _edge2_body(g2r, g2c, ea, cdr, w1r, w1c, w1e, b1, w2, b2, wc3, t_out):
    radial = cdr[:, 3:4]
    cdn = cdr[:, 0:3]
    lane0 = (lax.broadcasted_iota(jnp.int32, (1, 8), 1) == 0).astype(jnp.float32)
    eap = ea[...] + radial * lane0
    pre = (jnp.dot(g2r[...], w1r[...], preferred_element_type=jnp.float32)
           + jnp.dot(g2c[...], w1c[...], preferred_element_type=jnp.float32)
           + jnp.dot(eap, w1e[...], preferred_element_type=jnp.float32)
           + b1[...])
    s = _silu(pre)
    s = _silu(jnp.dot(s, w2[...], preferred_element_type=jnp.float32) + b2[...])
    t = jnp.sum(s * wc3[...], axis=1, keepdims=True)
    tr = cdn * jnp.tanh(t) * COORDS_RANGE
    t_out[...] = jnp.concatenate([tr, jnp.zeros((tr.shape[0], 13), jnp.float32)], axis=1)


def _node_body(h, p0, p1, wa, wb, b1, w2, b2, out):
    agg = (p0[...] + p1[...]) * (1.0 / NORM_FACTOR)
    pre = (jnp.dot(h[...], wa[...], preferred_element_type=jnp.float32)
           + jnp.dot(agg, wb[...], preferred_element_type=jnp.float32)
           + b1[...])
    u = _silu(pre)
    out[...] = h[...] + jnp.dot(u, w2[...], preferred_element_type=jnp.float32) + b2[...]


def _x_body(x, q0, q1, out):
    out[...] = x[...] + (q0[:, 0:3] + q1[:, 0:3]) * (1.0 / NORM_FACTOR)


def _blk(shape, pos=0):
    if pos is None:
        return pl.BlockSpec(shape, lambda i: (0,) * len(shape))
    return pl.BlockSpec(shape, lambda i: (i,) + (0,) * (len(shape) - 1))


def _edge1_call(gr, gc, eap, w1r, w1c, w1e, b1, w2, b2, watt, batt):
    grid = (E // EB,)
    return pl.pallas_call(
        _edge1_body,
        grid=grid,
        in_specs=[_blk((EB, D + 16)), _blk((EB, D + 16)), _blk((EB, 8)),
                  _blk((D, D), None), _blk((D, D), None), _blk((8, D), None),
                  _blk((1, D), None), _blk((D, D), None), _blk((1, D), None),
                  _blk((1, D), None), _blk((1, 1), None)],
        out_specs=[_blk((EB, D)), _blk((EB, 4))],
        out_shape=[jax.ShapeDtypeStruct((E, D), jnp.float32),
                   jax.ShapeDtypeStruct((E, 4), jnp.float32)],
    )(gr, gc, eap, w1r, w1c, w1e, b1, w2, b2, watt, batt)


def _edge2_call(g2r, g2c, eap, cdr, w1r, w1c, w1e, b1, w2, b2, wc3):
    grid = (E // EB,)
    return pl.pallas_call(
        _edge2_body,
        grid=grid,
        in_specs=[_blk((EB, D)), _blk((EB, D)), _blk((EB, 8)), _blk((EB, 4)),
                  _blk((D, D), None), _blk((D, D), None), _blk((8, D), None),
                  _blk((1, D), None), _blk((D, D), None), _blk((1, D), None),
                  _blk((1, D), None)],
        out_specs=_blk((EB, 16)),
        out_shape=jax.ShapeDtypeStruct((E, 16), jnp.float32),
    )(g2r, g2c, eap, cdr, w1r, w1c, w1e, b1, w2, b2, wc3)


def _node_call(h, p0, p1, wa, wb, b1, w2, b2):
    grid = (N // NB,)
    return pl.pallas_call(
        _node_body,
        grid=grid,
        in_specs=[_blk((NB, D)), _blk((NB, D)), _blk((NB, D)),
                  _blk((D, D), None), _blk((D, D), None), _blk((1, D), None),
                  _blk((D, D), None), _blk((1, D), None)],
        out_specs=_blk((NB, D)),
        out_shape=jax.ShapeDtypeStruct((N, D), jnp.float32),
    )(h, p0, p1, wa, wb, b1, w2, b2)


def _x_call(x, q0, q1):
    grid = (N // NB,)
    return pl.pallas_call(
        _x_body,
        grid=grid,
        in_specs=[_blk((NB, 3)), _blk((NB, 16)), _blk((NB, 16))],
        out_specs=_blk((NB, 3)),
        out_shape=jax.ShapeDtypeStruct((N, 3), jnp.float32),
    )(x, q0, q1)


def kernel(h, x, edge_index, edge_attr,
           W_m1, b_m1, W_m2, b_m2, W_att, b_att, W_u1, b_u1, W_u2, b_u2,
           W_c1, b_c1, W_c2, b_c2, W_c3):
    row = edge_index[0]
    col = edge_index[1]
    eap = jnp.pad(edge_attr, ((0, 0), (1, 3)))

    T1 = jnp.concatenate([h, x, jnp.zeros((N, 13), jnp.float32)], axis=1)
    gr, gc = _make_gather(D + 16)(T1, row, col)
    w1e = jnp.pad(W_m1[2 * D:], ((0, 3), (0, 0)))
    m, cdr = _edge1_call(gr, gc, eap,
                         W_m1[:D], W_m1[D:2 * D], w1e, b_m1.reshape(1, D),
                         W_m2, b_m2.reshape(1, D),
                         W_att.reshape(1, D), b_att.reshape(1, 1))
    part = _make_scatter(D)(m, row, jnp.zeros((RPT, D), jnp.float32))

    hn = _node_call(h, part[:N], part[N:],
                    W_u1[:D], W_u1[D:], b_u1.reshape(1, D),
                    W_u2, b_u2.reshape(1, D))

    g2r, g2c = _make_gather(D)(hn, row, col)
    wc1e = jnp.pad(W_c1[2 * D:], ((0, 3), (0, 0)))
    trans = _edge2_call(g2r, g2c, eap, cdr,
                        W_c1[:D], W_c1[D:2 * D], wc1e, b_c1.reshape(1, D),
                        W_c2, b_c2.reshape(1, D), W_c3.reshape(1, D))
    q = _make_scatter(16)(trans, row, jnp.zeros((RPT, 16), jnp.float32))
    xn = _x_call(x, q[:N], q[N:])
    return (hn, xn)


# speedup vs baseline: 2.0482x; 2.0482x over previous
import functools

import jax
import jax.numpy as jnp
from jax import lax
from jax.experimental import pallas as pl
from jax.experimental.pallas import tpu as pltpu
from jax.experimental.pallas import tpu_sc as plsc

N = 10000
E = 320000
D = 128
NORM_FACTOR = 100.0
COORDS_RANGE = 15.0
NORM_CONSTANT = 1.0

NC = 2
NS = 16
NW = NC * NS
EPW = E // NW
C = 80
NCH = EPW // C
RPT = N // NS

EB = 2000
NB = 1000


def _silu(v):
    return v * jax.nn.sigmoid(v)


def _make_gather(Dt):
    mesh = plsc.VectorSubcoreMesh(core_axis_name="c", subcore_axis_name="s")

    @functools.partial(
        pl.kernel,
        out_type=[jax.ShapeDtypeStruct((E, Dt), jnp.float32),
                  jax.ShapeDtypeStruct((E, Dt), jnp.float32)],
        mesh=mesh,
        scratch_types=[pltpu.VMEM((C,), jnp.int32),
                       pltpu.VMEM((C,), jnp.int32),
                       pltpu.VMEM((C, Dt), jnp.float32),
                       pltpu.VMEM((C, Dt), jnp.float32),
                       pltpu.SemaphoreType.DMA,
                       pltpu.SemaphoreType.DMA],
        compiler_params=pltpu.CompilerParams(use_tc_tiling_on_sc=False),
    )
    def gk(tab, rowi, coli, out_r, out_c, ir, ic, br, bc, s1, s2):
        wid = lax.axis_index("s") * NC + lax.axis_index("c")
        base0 = wid * EPW

        def body(i, carry):
            base = base0 + i * C
            pltpu.sync_copy(rowi.at[pl.ds(base, C)], ir)
            pltpu.sync_copy(coli.at[pl.ds(base, C)], ic)
            cp1 = pltpu.async_copy(tab.at[ir], br, s1)
            cp2 = pltpu.async_copy(tab.at[ic], bc, s2)
            cp1.wait()
            cp2.wait()
            pltpu.sync_copy(br, out_r.at[pl.ds(base, C)])
            pltpu.sync_copy(bc, out_c.at[pl.ds(base, C)])
            return carry

        lax.fori_loop(0, NCH, body, 0)

    return gk


def _make_scatter(Dv, CH):
    mesh = plsc.VectorSubcoreMesh(core_axis_name="c", subcore_axis_name="s")

    @functools.partial(
        pl.kernel,
        out_type=jax.ShapeDtypeStruct((NC * N, Dv), jnp.float32),
        mesh=mesh,
        scratch_types=[pltpu.VMEM((C,), jnp.int32),
                       pltpu.VMEM((C, CH), jnp.float32),
                       pltpu.VMEM((RPT, CH), jnp.float32),
                       pltpu.VMEM_SHARED((N, CH), jnp.float32)],
        compiler_params=pltpu.CompilerParams(use_tc_tiling_on_sc=False),
    )
    def sk(vals, rowi, zer, out, ib, vb, zb, acc):
        cid = lax.axis_index("c")
        sid = lax.axis_index("s")
        base0 = (cid * NS + sid) * EPW
        myrows = pl.ds(sid * RPT, RPT)
        for ch in range(Dv // CH):
            cols = pl.ds(ch * CH, CH)
            pltpu.sync_copy(zer, zb)
            pltpu.sync_copy(zb, acc.at[myrows])
            plsc.subcore_barrier()

            def body(i, carry):
                base = base0 + i * C
                pltpu.sync_copy(rowi.at[pl.ds(base, C)], ib)
                pltpu.sync_copy(vals.at[pl.ds(base, C), cols], vb)
                pltpu.sync_copy(vb, acc.at[ib], add=True)
                return carry

            lax.fori_loop(0, NCH, body, 0)
            plsc.subcore_barrier()
            pltpu.sync_copy(acc.at[myrows], zb)
            pltpu.sync_copy(zb, out.at[pl.ds(cid * N + sid * RPT, RPT), cols])

    return sk


def _edge1_body(gr, gc, ea, w1r, w1c, w1e, b1, w2, b2, watt, batt, m_out, cdr_out):
    hr = gr[:, :D]
    hc = gc[:, :D]
    xr = gr[:, D:D + 3]
    xc = gc[:, D:D + 3]
    cd = xr - xc
    radial = jnp.sum(cd * cd, axis=1, keepdims=True)
    cdn = cd / (jnp.sqrt(radial + 1e-8) + NORM_CONSTANT)
    lane0 = (lax.broadcasted_iota(jnp.int32, (1, 8), 1) == 0).astype(jnp.float32)
    eap = ea[...] + radial * lane0
    pre = (jnp.dot(hr, w1r[...], preferred_element_type=jnp.float32)
           + jnp.dot(hc, w1c[...], preferred_element_type=jnp.float32)
           + jnp.dot(eap, w1e[...], preferred_element_type=jnp.float32)
           + b1[...])
    m = _silu(pre)
    m = _silu(jnp.dot(m, w2[...], preferred_element_type=jnp.float32) + b2[...])
    att = jax.nn.sigmoid(jnp.sum(m * watt[...], axis=1, keepdims=True) + batt[...])
    m_out[...] = m * att
    cdr_out[...] = jnp.concatenate([cdn, radial], axis=1)


def _edge2_body(g2r, g2c, ea, cdr, w1r, w1c, w1e, b1, w2, b2, wc3, t_out):
    radial = cdr[:, 3:4]
    cdn = cdr[:, 0:3]
    lane0 = (lax.broadcasted_iota(jnp.int32, (1, 8), 1) == 0).astype(jnp.float32)
    eap = ea[...] + radial * lane0
    pre = (jnp.dot(g2r[...], w1r[...], preferred_element_type=jnp.float32)
           + jnp.dot(g2c[...], w1c[...], preferred_element_type=jnp.float32)
           + jnp.dot(eap, w1e[...], preferred_element_type=jnp.float32)
           + b1[...])
    s = _silu(pre)
    s = _silu(jnp.dot(s, w2[...], preferred_element_type=jnp.float32) + b2[...])
    t = jnp.sum(s * wc3[...], axis=1, keepdims=True)
    tr = cdn * jnp.tanh(t) * COORDS_RANGE
    t_out[...] = jnp.concatenate([tr, jnp.zeros((tr.shape[0], 13), jnp.float32)], axis=1)


def _node_body(h, p0, p1, wa, wb, b1, w2, b2, out):
    agg = (p0[...] + p1[...]) * (1.0 / NORM_FACTOR)
    pre = (jnp.dot(h[...], wa[...], preferred_element_type=jnp.float32)
           + jnp.dot(agg, wb[...], preferred_element_type=jnp.float32)
           + b1[...])
    u = _silu(pre)
    out[...] = h[...] + jnp.dot(u, w2[...], preferred_element_type=jnp.float32) + b2[...]


def _x_body(x, q0, q1, out):
    out[...] = x[...] + (q0[:, 0:3] + q1[:, 0:3]) * (1.0 / NORM_FACTOR)


def _blk(shape, pos=0):
    if pos is None:
        return pl.BlockSpec(shape, lambda i: (0,) * len(shape))
    return pl.BlockSpec(shape, lambda i: (i,) + (0,) * (len(shape) - 1))


def _edge1_call(gr, gc, eap, w1r, w1c, w1e, b1, w2, b2, watt, batt):
    grid = (E // EB,)
    return pl.pallas_call(
        _edge1_body,
        grid=grid,
        in_specs=[_blk((EB, D + 16)), _blk((EB, D + 16)), _blk((EB, 8)),
                  _blk((D, D), None), _blk((D, D), None), _blk((8, D), None),
                  _blk((1, D), None), _blk((D, D), None), _blk((1, D), None),
                  _blk((1, D), None), _blk((1, 1), None)],
        out_specs=[_blk((EB, D)), _blk((EB, 4))],
        out_shape=[jax.ShapeDtypeStruct((E, D), jnp.float32),
                   jax.ShapeDtypeStruct((E, 4), jnp.float32)],
    )(gr, gc, eap, w1r, w1c, w1e, b1, w2, b2, watt, batt)


def _edge2_call(g2r, g2c, eap, cdr, w1r, w1c, w1e, b1, w2, b2, wc3):
    grid = (E // EB,)
    return pl.pallas_call(
        _edge2_body,
        grid=grid,
        in_specs=[_blk((EB, D)), _blk((EB, D)), _blk((EB, 8)), _blk((EB, 4)),
                  _blk((D, D), None), _blk((D, D), None), _blk((8, D), None),
                  _blk((1, D), None), _blk((D, D), None), _blk((1, D), None),
                  _blk((1, D), None)],
        out_specs=_blk((EB, 16)),
        out_shape=jax.ShapeDtypeStruct((E, 16), jnp.float32),
    )(g2r, g2c, eap, cdr, w1r, w1c, w1e, b1, w2, b2, wc3)


def _node_call(h, p0, p1, wa, wb, b1, w2, b2):
    grid = (N // NB,)
    return pl.pallas_call(
        _node_body,
        grid=grid,
        in_specs=[_blk((NB, D)), _blk((NB, D)), _blk((NB, D)),
                  _blk((D, D), None), _blk((D, D), None), _blk((1, D), None),
                  _blk((D, D), None), _blk((1, D), None)],
        out_specs=_blk((NB, D)),
        out_shape=jax.ShapeDtypeStruct((N, D), jnp.float32),
    )(h, p0, p1, wa, wb, b1, w2, b2)


def _x_call(x, q0, q1):
    grid = (N // NB,)
    return pl.pallas_call(
        _x_body,
        grid=grid,
        in_specs=[_blk((NB, 3)), _blk((NB, 16)), _blk((NB, 16))],
        out_specs=_blk((NB, 3)),
        out_shape=jax.ShapeDtypeStruct((N, 3), jnp.float32),
    )(x, q0, q1)


def kernel(h, x, edge_index, edge_attr,
           W_m1, b_m1, W_m2, b_m2, W_att, b_att, W_u1, b_u1, W_u2, b_u2,
           W_c1, b_c1, W_c2, b_c2, W_c3):
    row = edge_index[0]
    col = edge_index[1]
    eap = jnp.pad(edge_attr, ((0, 0), (1, 3)))

    T1 = jnp.concatenate([h, x, jnp.zeros((N, 13), jnp.float32)], axis=1)
    gr, gc = _make_gather(D + 16)(T1, row, col)
    w1e = jnp.pad(W_m1[2 * D:], ((0, 3), (0, 0)))
    m, cdr = _edge1_call(gr, gc, eap,
                         W_m1[:D], W_m1[D:2 * D], w1e, b_m1.reshape(1, D),
                         W_m2, b_m2.reshape(1, D),
                         W_att.reshape(1, D), b_att.reshape(1, 1))
    part = _make_scatter(D, 64)(m, row, jnp.zeros((RPT, 64), jnp.float32))

    hn = _node_call(h, part[:N], part[N:],
                    W_u1[:D], W_u1[D:], b_u1.reshape(1, D),
                    W_u2, b_u2.reshape(1, D))

    g2r, g2c = _make_gather(D)(hn, row, col)
    wc1e = jnp.pad(W_c1[2 * D:], ((0, 3), (0, 0)))
    trans = _edge2_call(g2r, g2c, eap, cdr,
                        W_c1[:D], W_c1[D:2 * D], wc1e, b_c1.reshape(1, D),
                        W_c2, b_c2.reshape(1, D), W_c3.reshape(1, D))
    q = _make_scatter(16, 16)(trans, row, jnp.zeros((RPT, 16), jnp.float32))
    xn = _x_call(x, q[:N], q[N:])
    return (hn, xn)
